# Initial kernel scaffold; baseline (speedup 1.0000x reference)
import jax
import jax.numpy as jnp
from jax.experimental import pallas as pl


def kernel(z):
    raise NotImplementedError("write your pallas kernel here")


# speedup vs baseline: 4.6537x; 4.6537x over previous
import jax
import jax.numpy as jnp
from jax.experimental import pallas as pl

_LANES = 128


def _rev_body(z_ref, o_ref):
    bm, width = z_ref.shape
    nchunk = width // _LANES
    idx = (_LANES - 1) - jax.lax.broadcasted_iota(jnp.int32, (bm, _LANES), 1)
    x = z_ref[...]
    for a in range(nchunk):
        chunk = x[:, a * _LANES:(a + 1) * _LANES]
        rev = jnp.take_along_axis(chunk, idx, axis=1, mode="promise_in_bounds")
        o_ref[:, (nchunk - 1 - a) * _LANES:(nchunk - a) * _LANES] = rev


def kernel(z):
    B, Z = z.shape
    BM = 512
    return pl.pallas_call(
        _rev_body,
        grid=(B // BM,),
        in_specs=[pl.BlockSpec((BM, Z), lambda i: (i, 0))],
        out_specs=pl.BlockSpec((BM, Z), lambda i: (i, 0)),
        out_shape=jax.ShapeDtypeStruct((B, Z), z.dtype),
    )(z)
